# Initial kernel scaffold; baseline (speedup 1.0000x reference)
import jax
import jax.numpy as jnp
from jax.experimental import pallas as pl


def kernel(attn, feat):
    raise NotImplementedError("write your pallas kernel here")


# speedup vs baseline: 175.9775x; 175.9775x over previous
import jax
import jax.numpy as jnp
from jax.experimental import pallas as pl
from jax.experimental.pallas import tpu as pltpu

_P_THR = 0.55
_C_THR = 0.7
_W_FEAT = 1.0
_W_ATTN = 1.0


def _consistence_kernel(attn_ref, feat_ref, out_ref, acc_ref):
    b = pl.program_id(0)
    nb = pl.num_programs(0)

    @pl.when(b == 0)
    def _init():
        acc_ref[0] = 0.0
        acc_ref[1] = 0.0
        acc_ref[2] = 0.0

    a = attn_ref[0]
    T = a.shape[1]
    nseg = (T + 1) // 2

    pred = a > _P_THR
    pred_f = jnp.where(pred, 1.0, 0.0)
    r = jax.lax.broadcasted_iota(jnp.int32, (T, T), 0)
    c = jax.lax.broadcasted_iota(jnp.int32, (T, T), 1)
    shift = jnp.where(r + 1 == c, 1.0, 0.0)
    triu = jnp.where(r <= c, 1.0, 0.0)
    prev_f = jnp.dot(pred_f, shift, preferred_element_type=jnp.float32)
    start_f = pred_f * (1.0 - prev_f)
    cum = jnp.dot(start_f, triu, preferred_element_type=jnp.float32)
    seg = cum.astype(jnp.int32) - 1

    row_ids = jax.lax.broadcasted_iota(jnp.int32, (nseg, T), 0)
    m = jnp.where((row_ids == seg) & pred, 1.0, 0.0)
    rep_f = jnp.where(a > _C_THR, 1.0, 0.0)
    m_rep = m * rep_f

    counts = jnp.sum(m, axis=1, keepdims=True)
    rep_counts = jnp.sum(m_rep, axis=1, keepdims=True)
    sum_a = jnp.sum(m * a, axis=1, keepdims=True)
    sum_a2 = jnp.sum(m * (a * a), axis=1, keepdims=True)

    valid = counts > 0.0
    counts_safe = jnp.where(valid, counts, 1.0)
    mean_a = sum_a / counts_safe
    var = sum_a2 / counts_safe - mean_a * mean_a
    nprop = jnp.sum(jnp.where(valid, 1.0, 0.0))
    video_loss = jnp.sum(jnp.where(valid, var, 0.0))
    attn_contrib = jnp.where(nprop > 0.0, video_loss / jnp.maximum(nprop, 1.0), 0.0)

    fi = feat_ref[0]
    d = fi.shape[1]
    sum_feat = jnp.dot(m, fi, preferred_element_type=jnp.float32)
    sum_rep = jnp.dot(m_rep, fi, preferred_element_type=jnp.float32)

    has_rep = valid & (rep_counts > 0.0)
    rep_safe = jnp.where(has_rep, rep_counts, 1.0)
    diff = sum_feat / counts_safe - sum_rep / rep_safe
    mse = jnp.sum(diff * diff, axis=1, keepdims=True) / d
    feat_contrib = jnp.sum(jnp.where(has_rep, mse, 0.0))
    cnt_contrib = jnp.sum(jnp.where(has_rep, 1.0, 0.0))

    acc_ref[0] += feat_contrib
    acc_ref[1] += cnt_contrib
    acc_ref[2] += attn_contrib

    @pl.when(b == nb - 1)
    def _fin():
        fls = acc_ref[0]
        fc = acc_ref[1]
        feat_loss = jnp.where(fc > 0.0, fls / jnp.maximum(fc, 1.0), fls)
        out_ref[0, 0] = _W_FEAT * feat_loss + _W_ATTN * acc_ref[2] / nb


def kernel(attn, feat):
    B, T, _ = attn.shape
    D = feat.shape[2]
    attn2 = attn.reshape(B, 1, T)
    out = pl.pallas_call(
        _consistence_kernel,
        grid=(B,),
        in_specs=[
            pl.BlockSpec((1, 1, T), lambda b: (b, 0, 0)),
            pl.BlockSpec((1, T, D), lambda b: (b, 0, 0)),
        ],
        out_specs=pl.BlockSpec(memory_space=pltpu.SMEM),
        out_shape=jax.ShapeDtypeStruct((1, 1), jnp.float32),
        scratch_shapes=[pltpu.SMEM((3,), jnp.float32)],
    )(attn2, feat)
    return out[0, 0]
